# baseline (speedup 1.0000x reference)
import functools

import jax
import jax.numpy as jnp
from jax import lax
from jax.experimental import pallas as pl
from jax.experimental.pallas import tpu as pltpu
from jax.experimental.pallas import tpu_sc as plsc

N, D = 16384, 64

_info = plsc.get_sparse_core_info()
NC, NS, L = _info.num_cores, _info.num_subcores, _info.num_lanes
NW = NC * NS
SN = 4096
RS = SN // NW
TPC = RS // L
KI = 32
KO = D // KI

TN = N - SN
TBLK = 1024

_mesh = plsc.VectorSubcoreMesh(core_axis_name="c", subcore_axis_name="s")


@functools.partial(
    pl.kernel,
    mesh=_mesh,
    out_type=jax.ShapeDtypeStruct((SN,), jnp.float32),
    compiler_params=pltpu.CompilerParams(needs_layout_passes=False),
    scratch_types=[
        pltpu.VMEM((D, RS), jnp.float32),
        pltpu.VMEM((D, RS), jnp.float32),
        pltpu.VMEM((RS,), jnp.float32),
        pltpu.SemaphoreType.DMA,
        pltpu.SemaphoreType.DMA,
    ],
)
def _rowdot_sc(gut_hbm, git_hbm, out_hbm, gu_v, gi_v, out_v, su, si):
    wid = lax.axis_index("s") * NC + lax.axis_index("c")
    base = wid * RS
    pltpu.async_copy(gut_hbm.at[:, pl.ds(base, RS)], gu_v, su)
    pltpu.async_copy(git_hbm.at[:, pl.ds(base, RS)], gi_v, si)
    pltpu.make_async_copy(gut_hbm.at[:, pl.ds(0, RS)], gu_v, su).wait()
    pltpu.make_async_copy(git_hbm.at[:, pl.ds(0, RS)], gi_v, si).wait()

    def tbody(t, carry):
        s = pl.multiple_of(t * L, L)

        def kbody(kk, acc):
            k0 = kk * KI
            for k in range(KI):
                acc = acc + (gu_v[k0 + k, pl.ds(s, L)]
                             * gi_v[k0 + k, pl.ds(s, L)])
            return acc

        acc = lax.fori_loop(0, KO, kbody, jnp.zeros((L,), jnp.float32))
        out_v[pl.ds(s, L)] = acc
        return carry

    lax.fori_loop(0, TPC, tbody, 0)
    pltpu.sync_copy(out_v, out_hbm.at[pl.ds(base, RS)])


def _rowdot_tc_body(gu_ref, gi_ref, out_ref):
    out_ref[...] = jnp.sum(gu_ref[...] * gi_ref[...], axis=0)


_rowdot_tc = pl.pallas_call(
    _rowdot_tc_body,
    grid=(TN // TBLK,),
    in_specs=[
        pl.BlockSpec((D, TBLK), lambda g: (0, g + SN // TBLK)),
        pl.BlockSpec((D, TBLK), lambda g: (0, g + SN // TBLK)),
    ],
    out_specs=pl.BlockSpec((TBLK,), lambda g: (g,)),
    out_shape=jax.ShapeDtypeStruct((TN,), jnp.float32),
)


def kernel(gu, gi):
    gut, git = gu.T, gi.T
    sc_out = _rowdot_sc(gut, git)
    tc_out = _rowdot_tc(gut, git)
    return jnp.concatenate([sc_out, tc_out])


# speedup vs baseline: 1.0809x; 1.0809x over previous
import functools

import jax
import jax.numpy as jnp
from jax import lax
from jax.experimental import pallas as pl
from jax.experimental.pallas import tpu as pltpu
from jax.experimental.pallas import tpu_sc as plsc

N, D = 16384, 64

_info = plsc.get_sparse_core_info()
NC, NS, L = _info.num_cores, _info.num_subcores, _info.num_lanes
NW = NC * NS
SN = 4096
RS = SN // NW
TPC = RS // L
KI = 32
KO = D // KI

TN = N - SN
TBLK = 2048

_mesh = plsc.VectorSubcoreMesh(core_axis_name="c", subcore_axis_name="s")


@functools.partial(
    pl.kernel,
    mesh=_mesh,
    out_type=jax.ShapeDtypeStruct((SN,), jnp.float32),
    compiler_params=pltpu.CompilerParams(needs_layout_passes=False),
    scratch_types=[
        pltpu.VMEM((D, RS), jnp.float32),
        pltpu.VMEM((D, RS), jnp.float32),
        pltpu.VMEM((RS,), jnp.float32),
        pltpu.SemaphoreType.DMA,
        pltpu.SemaphoreType.DMA,
    ],
)
def _rowdot_sc(gut_hbm, git_hbm, out_hbm, gu_v, gi_v, out_v, su, si):
    wid = lax.axis_index("s") * NC + lax.axis_index("c")
    base = wid * RS
    pltpu.async_copy(gut_hbm.at[:, pl.ds(base, RS)], gu_v, su)
    pltpu.async_copy(git_hbm.at[:, pl.ds(base, RS)], gi_v, si)
    pltpu.make_async_copy(gut_hbm.at[:, pl.ds(0, RS)], gu_v, su).wait()
    pltpu.make_async_copy(git_hbm.at[:, pl.ds(0, RS)], gi_v, si).wait()

    def tbody(t, carry):
        s = pl.multiple_of(t * L, L)

        def kbody(kk, acc):
            k0 = kk * KI
            for k in range(KI):
                acc = acc + (gu_v[k0 + k, pl.ds(s, L)]
                             * gi_v[k0 + k, pl.ds(s, L)])
            return acc

        acc = lax.fori_loop(0, KO, kbody, jnp.zeros((L,), jnp.float32))
        out_v[pl.ds(s, L)] = acc
        return carry

    lax.fori_loop(0, TPC, tbody, 0)
    pltpu.sync_copy(out_v, out_hbm.at[pl.ds(base, RS)])


def _rowdot_tc_body(gu_ref, gi_ref, out_ref):
    out_ref[...] = jnp.sum(gu_ref[...] * gi_ref[...], axis=0)


_rowdot_tc = pl.pallas_call(
    _rowdot_tc_body,
    grid=(TN // TBLK,),
    in_specs=[
        pl.BlockSpec((D, TBLK), lambda g: (0, g + SN // TBLK)),
        pl.BlockSpec((D, TBLK), lambda g: (0, g + SN // TBLK)),
    ],
    out_specs=pl.BlockSpec((TBLK,), lambda g: (g,)),
    out_shape=jax.ShapeDtypeStruct((TN,), jnp.float32),
)


def kernel(gu, gi):
    gut, git = gu.T, gi.T
    sc_out = _rowdot_sc(gut, git)
    tc_out = _rowdot_tc(gut, git)
    return jnp.concatenate([sc_out, tc_out])


# speedup vs baseline: 1.0873x; 1.0060x over previous
import functools

import jax
import jax.numpy as jnp
from jax import lax
from jax.experimental import pallas as pl
from jax.experimental.pallas import tpu as pltpu
from jax.experimental.pallas import tpu_sc as plsc

N, D = 16384, 64

_info = plsc.get_sparse_core_info()
NC, NS, L = _info.num_cores, _info.num_subcores, _info.num_lanes
NW = NC * NS
SN = 4096
RS = SN // NW
TPC = RS // L
KI = 32
KO = D // KI

TN = N - SN
TBLK = 4096

_mesh = plsc.VectorSubcoreMesh(core_axis_name="c", subcore_axis_name="s")


@functools.partial(
    pl.kernel,
    mesh=_mesh,
    out_type=jax.ShapeDtypeStruct((SN,), jnp.float32),
    compiler_params=pltpu.CompilerParams(needs_layout_passes=False),
    scratch_types=[
        pltpu.VMEM((D, RS), jnp.float32),
        pltpu.VMEM((D, RS), jnp.float32),
        pltpu.VMEM((RS,), jnp.float32),
        pltpu.SemaphoreType.DMA,
        pltpu.SemaphoreType.DMA,
    ],
)
def _rowdot_sc(gut_hbm, git_hbm, out_hbm, gu_v, gi_v, out_v, su, si):
    wid = lax.axis_index("s") * NC + lax.axis_index("c")
    base = wid * RS
    pltpu.async_copy(gut_hbm.at[:, pl.ds(base, RS)], gu_v, su)
    pltpu.async_copy(git_hbm.at[:, pl.ds(base, RS)], gi_v, si)
    pltpu.make_async_copy(gut_hbm.at[:, pl.ds(0, RS)], gu_v, su).wait()
    pltpu.make_async_copy(git_hbm.at[:, pl.ds(0, RS)], gi_v, si).wait()

    def tbody(t, carry):
        s = pl.multiple_of(t * L, L)

        def kbody(kk, acc):
            k0 = kk * KI
            for k in range(KI):
                acc = acc + (gu_v[k0 + k, pl.ds(s, L)]
                             * gi_v[k0 + k, pl.ds(s, L)])
            return acc

        acc = lax.fori_loop(0, KO, kbody, jnp.zeros((L,), jnp.float32))
        out_v[pl.ds(s, L)] = acc
        return carry

    lax.fori_loop(0, TPC, tbody, 0)
    pltpu.sync_copy(out_v, out_hbm.at[pl.ds(base, RS)])


def _rowdot_tc_body(gu_ref, gi_ref, out_ref):
    out_ref[...] = jnp.sum(gu_ref[...] * gi_ref[...], axis=0)


_rowdot_tc = pl.pallas_call(
    _rowdot_tc_body,
    grid=(TN // TBLK,),
    in_specs=[
        pl.BlockSpec((D, TBLK), lambda g: (0, g + SN // TBLK)),
        pl.BlockSpec((D, TBLK), lambda g: (0, g + SN // TBLK)),
    ],
    out_specs=pl.BlockSpec((TBLK,), lambda g: (g + SN // TBLK)),
    out_shape=jax.ShapeDtypeStruct((N,), jnp.float32),
)


def kernel(gu, gi):
    gut, git = gu.T, gi.T
    sc_out = _rowdot_sc(gut, git)
    tc_full = _rowdot_tc(gut, git)
    return lax.dynamic_update_slice(tc_full, sc_out, (0,))
